# baseline (speedup 1.0000x reference)
import functools
import jax
import jax.numpy as jnp
from jax import lax
from jax.experimental import pallas as pl
from jax.experimental.pallas import tpu as pltpu
from jax.experimental.pallas import tpu_sc as plsc

_NC = 2
_NS = 16
_NW = _NC * _NS
_CHUNK = 200
_LANES = 16


def _make_sc_kernel(t, jn, d, max_len, ncv):
    n_chunks = jn // _CHUNK
    n_tasks = t * n_chunks
    k_max = (n_tasks + _NW - 1) // _NW
    gpc = _CHUNK // ncv
    half = max_len // 2
    nv = d // _LANES

    mesh = plsc.VectorSubcoreMesh(core_axis_name="c", subcore_axis_name="s")

    @functools.partial(
        pl.kernel,
        mesh=mesh,
        out_type=jax.ShapeDtypeStruct((t, jn, d), jnp.float32),
        scratch_types=[
            pltpu.VMEM((max_len, d), jnp.float32),
            pltpu.VMEM((_CHUNK, d), jnp.float32),
            pltpu.VMEM((_CHUNK, d), jnp.float32),
            pltpu.SemaphoreType.DMA,
            pltpu.SemaphoreType.DMA,
            pltpu.SemaphoreType.DMA,
            pltpu.SemaphoreType.DMA,
        ],
    )
    def sc_kernel(x_hbm, e_hbm, out_hbm, e_v, buf0, buf1,
                  lsem0, lsem1, ssem0, ssem1):
        wid = lax.axis_index("c") * _NS + lax.axis_index("s")
        pltpu.sync_copy(e_hbm, e_v)

        bufs = (buf0, buf1)
        lsems = (lsem0, lsem1)
        ssems = (ssem0, ssem1)

        def task(k):
            return wid + k * _NW

        def load(k, b):
            tsk = task(k)
            i = tsk // n_chunks
            j0 = (tsk % n_chunks) * _CHUNK
            pltpu.async_copy(x_hbm.at[i, pl.ds(j0, _CHUNK), :], bufs[b],
                             lsems[b])

        def store(k, b):
            tsk = task(k)
            i = tsk // n_chunks
            j0 = (tsk % n_chunks) * _CHUNK
            pltpu.async_copy(bufs[b], out_hbm.at[i, pl.ds(j0, _CHUNK), :],
                             ssems[b])

        def wait_load(b):
            pltpu.make_async_copy(x_hbm.at[0, pl.ds(0, _CHUNK), :], bufs[b],
                                  lsems[b]).wait()

        def wait_store(b):
            pltpu.make_async_copy(bufs[b], out_hbm.at[0, pl.ds(0, _CHUNK), :],
                                  ssems[b]).wait()

        def compute(k, b):
            tsk = task(k)
            i = tsk // n_chunks
            g0 = (tsk % n_chunks) * gpc
            buf = bufs[b]

            def group_body(g_off, carry):
                gidx = jnp.minimum(jnp.maximum(g0 + g_off - i + half, 0),
                                   max_len - 1)
                for j in range(ncv):
                    r = g_off * ncv + j
                    for v in range(nv):
                        sl = pl.ds(v * _LANES, _LANES)
                        buf[r, sl] = buf[r, sl] + e_v[gidx, sl]
                return carry

            lax.fori_loop(0, gpc, group_body, 0, unroll=False)

        load(0, 0)

        def ring_body(kk, carry):
            for b in (0, 1):
                k = kk * 2 + b
                valid = task(k) < n_tasks

                @pl.when(valid)
                def _():
                    @pl.when(task(k + 1) < n_tasks)
                    def _():
                        @pl.when(k >= 1)
                        def _():
                            wait_store(1 - b)

                        load(k + 1, 1 - b)

                    wait_load(b)
                    compute(k, b)
                    store(k, b)
            return carry

        lax.fori_loop(0, (k_max + 2) // 2, ring_body, 0, unroll=False)
        wait_store(0)
        wait_store(1)

    return sc_kernel


def kernel(x, embedding_weight):
    t = x.shape[0]
    jn = x.shape[1]
    d = x.shape[2]
    max_len = embedding_weight.shape[0]
    ncv = jn // t
    return _make_sc_kernel(t, jn, d, max_len, ncv)(x, embedding_weight)


# speedup vs baseline: 3.0608x; 3.0608x over previous
import functools
import jax
import jax.numpy as jnp
from jax import lax
from jax.experimental import pallas as pl
from jax.experimental.pallas import tpu as pltpu
from jax.experimental.pallas import tpu_sc as plsc

_BI = 8
_SPLIT = 160

_NC = 2
_NS = 16
_NW = _NC * _NS
_CHUNK = 200
_LANES = 16


def _tc_body(e_ref, x_ref, o_ref, rep_ref):
    i0 = pl.program_id(0) * _BI
    max_len, d = e_ref.shape
    n = rep_ref.shape[1]
    half = max_len // 2

    @pl.when(i0 == 0)
    def _build_rep():
        j = jax.lax.broadcasted_iota(jnp.int32, rep_ref.shape, 0)
        g = jax.lax.broadcasted_iota(jnp.int32, rep_ref.shape, 1)
        ncv = rep_ref.shape[0] // n
        rep_ref[...] = (j // ncv == g).astype(jnp.bfloat16)

    r = jax.lax.broadcasted_iota(jnp.int32, (n, max_len), 0)
    k = jax.lax.broadcasted_iota(jnp.int32, (n, max_len), 1)
    for bi in range(_BI):
        idx = jnp.clip(r - (i0 + bi) + half, 0, max_len - 1)
        onehot = (k == idx).astype(jnp.float32)
        s = jnp.dot(onehot, e_ref[...], preferred_element_type=jnp.float32)
        addend = jnp.dot(rep_ref[...], s.astype(jnp.bfloat16),
                         preferred_element_type=jnp.float32)
        o_ref[bi, :, :] = x_ref[bi, :, :] + addend


def _make_sc_kernel(t, jn, d, max_len, ncv, i_base, n_rows):
    n_chunks = jn // _CHUNK
    n_tasks = n_rows * n_chunks
    k_max = (n_tasks + _NW - 1) // _NW
    gpc = _CHUNK // ncv
    half = max_len // 2
    nv = d // _LANES

    mesh = plsc.VectorSubcoreMesh(core_axis_name="c", subcore_axis_name="s")

    @functools.partial(
        pl.kernel,
        mesh=mesh,
        out_type=jax.ShapeDtypeStruct((n_rows, jn, d), jnp.float32),
        scratch_types=[
            pltpu.VMEM((max_len, d), jnp.float32),
            pltpu.VMEM((_CHUNK, d), jnp.float32),
            pltpu.VMEM((_CHUNK, d), jnp.float32),
            pltpu.SemaphoreType.DMA,
            pltpu.SemaphoreType.DMA,
            pltpu.SemaphoreType.DMA,
            pltpu.SemaphoreType.DMA,
        ],
    )
    def sc_kernel(x_hbm, e_hbm, out_hbm, e_v, buf0, buf1,
                  lsem0, lsem1, ssem0, ssem1):
        wid = lax.axis_index("c") * _NS + lax.axis_index("s")
        pltpu.sync_copy(e_hbm, e_v)

        bufs = (buf0, buf1)
        lsems = (lsem0, lsem1)
        ssems = (ssem0, ssem1)

        def task(k):
            return wid + k * _NW

        def load(k, b):
            tsk = task(k)
            i = tsk // n_chunks
            j0 = (tsk % n_chunks) * _CHUNK
            pltpu.async_copy(x_hbm.at[i_base + i, pl.ds(j0, _CHUNK), :],
                             bufs[b], lsems[b])

        def store(k, b):
            tsk = task(k)
            i = tsk // n_chunks
            j0 = (tsk % n_chunks) * _CHUNK
            pltpu.async_copy(bufs[b], out_hbm.at[i, pl.ds(j0, _CHUNK), :],
                             ssems[b])

        def wait_load(b):
            pltpu.make_async_copy(x_hbm.at[0, pl.ds(0, _CHUNK), :], bufs[b],
                                  lsems[b]).wait()

        def wait_store(b):
            pltpu.make_async_copy(bufs[b], out_hbm.at[0, pl.ds(0, _CHUNK), :],
                                  ssems[b]).wait()

        def compute(k, b):
            tsk = task(k)
            i = i_base + tsk // n_chunks
            g0 = (tsk % n_chunks) * gpc
            buf = bufs[b]

            def group_body(g_off, carry):
                gidx = jnp.minimum(jnp.maximum(g0 + g_off - i + half, 0),
                                   max_len - 1)
                for j in range(ncv):
                    r = g_off * ncv + j
                    for v in range(nv):
                        sl = pl.ds(v * _LANES, _LANES)
                        buf[r, sl] = buf[r, sl] + e_v[gidx, sl]
                return carry

            lax.fori_loop(0, gpc, group_body, 0, unroll=False)

        load(0, 0)

        def ring_body(kk, carry):
            for b in (0, 1):
                k = kk * 2 + b
                valid = task(k) < n_tasks

                @pl.when(valid)
                def _():
                    @pl.when(task(k + 1) < n_tasks)
                    def _():
                        @pl.when(k >= 1)
                        def _():
                            wait_store(1 - b)

                        load(k + 1, 1 - b)

                    wait_load(b)
                    compute(k, b)
                    store(k, b)
            return carry

        lax.fori_loop(0, (k_max + 2) // 2, ring_body, 0, unroll=False)
        wait_store(0)
        wait_store(1)

    return sc_kernel


def kernel(x, embedding_weight):
    t = x.shape[0]
    jn = x.shape[1]
    d = x.shape[2]
    max_len = embedding_weight.shape[0]
    ncv = jn // t

    tc_out = pl.pallas_call(
        _tc_body,
        grid=(_SPLIT // _BI,),
        in_specs=[
            pl.BlockSpec((max_len, d), lambda i: (0, 0)),
            pl.BlockSpec((_BI, jn, d), lambda i: (i, 0, 0)),
        ],
        out_specs=pl.BlockSpec((_BI, jn, d), lambda i: (i, 0, 0)),
        out_shape=jax.ShapeDtypeStruct(x.shape, x.dtype),
        scratch_shapes=[pltpu.VMEM((jn, t), jnp.bfloat16)],
    )(embedding_weight, x)

    sc_out = _make_sc_kernel(t, jn, d, max_len, ncv, _SPLIT, t - _SPLIT)(
        x, embedding_weight)

    return lax.dynamic_update_slice(tc_out, sc_out, (_SPLIT, 0, 0))


# speedup vs baseline: 4.2136x; 1.3766x over previous
import jax
import jax.numpy as jnp
from jax.experimental import pallas as pl
from jax.experimental.pallas import tpu as pltpu


_BI = 8


def _body(e_ref, x_ref, o_ref, rep_ref):
    i0 = pl.program_id(0) * _BI
    max_len, d = e_ref.shape
    n = rep_ref.shape[1]
    half = max_len // 2

    @pl.when(i0 == 0)
    def _build_rep():
        j = jax.lax.broadcasted_iota(jnp.int32, rep_ref.shape, 0)
        g = jax.lax.broadcasted_iota(jnp.int32, rep_ref.shape, 1)
        ncv = rep_ref.shape[0] // n
        rep_ref[...] = (j // ncv == g).astype(jnp.bfloat16)

    r = jax.lax.broadcasted_iota(jnp.int32, (n, max_len), 0)
    k = jax.lax.broadcasted_iota(jnp.int32, (n, max_len), 1)
    for bi in range(_BI):
        idx = jnp.clip(r - (i0 + bi) + half, 0, max_len - 1)
        onehot = (k == idx).astype(jnp.float32)
        s = jnp.dot(onehot, e_ref[...], preferred_element_type=jnp.float32)
        addend = jnp.dot(rep_ref[...], s.astype(jnp.bfloat16),
                         preferred_element_type=jnp.float32)
        o_ref[bi, :, :] = x_ref[bi, :, :] + addend


def kernel(x, embedding_weight):
    t = x.shape[0]
    jn = x.shape[1]
    d = x.shape[2]
    max_len = embedding_weight.shape[0]

    return pl.pallas_call(
        _body,
        grid=(t // _BI,),
        in_specs=[
            pl.BlockSpec((max_len, d), lambda i: (0, 0)),
            pl.BlockSpec((_BI, jn, d), lambda i: (i, 0, 0)),
        ],
        out_specs=pl.BlockSpec((_BI, jn, d), lambda i: (i, 0, 0)),
        out_shape=jax.ShapeDtypeStruct(x.shape, x.dtype),
        scratch_shapes=[pltpu.VMEM((jn, t), jnp.bfloat16)],
    )(embedding_weight, x)
